# baseline (speedup 1.0000x reference)
import functools

import jax
import jax.numpy as jnp
import numpy as np
from jax import lax
from jax.experimental import pallas as pl
from jax.experimental.pallas import tpu as pltpu
from jax.experimental.pallas import tpu_sc as plsc

N = 10000
E = 320000
D = 128
H = 128
EE = 16
DEPTH = 2

K = 64
NBLK = E // K
NCORE = 2
NSUB = 16
NW = NCORE * NSUB
ROWS_PER_SUB = 624

_F32 = jnp.float32
_BF16 = jnp.bfloat16
_C1 = 1.5957691216057308
_C2 = _C1 * 0.044715
_mC1 = -_C1
_mC2 = -_C2
_LOG2E = 1.4426950408889634
_mC1e = _mC1 * _LOG2E
_mC2e = _mC2 * _LOG2E

_PERMC = np.concatenate([
    np.concatenate([np.arange(32 * g, 32 * g + 16) for g in range(H // 32)]),
    np.concatenate([np.arange(32 * g + 16, 32 * g + 32) for g in range(H // 32)]),
]).astype(np.int32)


def _ef_body(ea, w1, b1, w2, b2, out):
    h = jax.nn.gelu(jnp.dot(ea[...], w1[...], preferred_element_type=_F32) + b1[...])
    out[...] = jnp.dot(h, w2[...], preferred_element_type=_F32) + b2[...]


def _compute_ef(edge_attr, w1, b1, w2, b2):
    BE = 4000
    return pl.pallas_call(
        _ef_body,
        grid=(E // BE,),
        in_specs=[
            pl.BlockSpec((BE, 4), lambda i: (i, 0)),
            pl.BlockSpec((4, EE), lambda i: (0, 0)),
            pl.BlockSpec((1, EE), lambda i: (0, 0)),
            pl.BlockSpec((EE, EE), lambda i: (0, 0)),
            pl.BlockSpec((1, EE), lambda i: (0, 0)),
        ],
        out_specs=pl.BlockSpec((BE, EE), lambda i: (i, 0)),
        out_shape=jax.ShapeDtypeStruct((E, EE), _F32),
    )(edge_attr, w1, b1.reshape(1, EE), w2, b2.reshape(1, EE))


BE = 4000


def _c_body(ef, wc, out):
    m = jnp.dot(ef[...], wc[...], preferred_element_type=_F32)
    lo = lax.bitcast_convert_type(m[:, :H // 2].astype(_BF16), jnp.uint16)
    hi = lax.bitcast_convert_type(m[:, H // 2:].astype(_BF16), jnp.uint16)
    packed = lo.astype(jnp.uint32) | (hi.astype(jnp.uint32) << jnp.uint32(16))
    out[...] = lax.bitcast_convert_type(packed, jnp.int32)


def _compute_c(ef, wc):
    return pl.pallas_call(
        _c_body,
        grid=(E // BE,),
        in_specs=[
            pl.BlockSpec((BE, EE), lambda i: (i, 0)),
            pl.BlockSpec((EE, H), lambda i: (0, 0)),
        ],
        out_specs=pl.BlockSpec((BE, H // 2), lambda i: (i, 0)),
        out_shape=jax.ShapeDtypeStruct((E, H // 2), jnp.int32),
    )(ef, wc)


BN = 2000


def _node_first_body(x, wf, bf, wa, wb, bm, h_out, a_out, b_out):
    h = jax.nn.gelu(jnp.dot(x[...], wf[...], preferred_element_type=_F32) + bf[...])
    h_out[...] = h
    a_out[...] = jnp.dot(h, wa[...], preferred_element_type=_F32)
    b_out[...] = jnp.dot(h, wb[...], preferred_element_type=_F32) + bm[...]


def _node_first(x, wf, bf, wa, wb, bm):
    return pl.pallas_call(
        _node_first_body,
        grid=(N // BN,),
        in_specs=[
            pl.BlockSpec((BN, D), lambda i: (i, 0)),
            pl.BlockSpec((D, H), lambda i: (0, 0)),
            pl.BlockSpec((1, H), lambda i: (0, 0)),
            pl.BlockSpec((H, H), lambda i: (0, 0)),
            pl.BlockSpec((H, H), lambda i: (0, 0)),
            pl.BlockSpec((1, H), lambda i: (0, 0)),
        ],
        out_specs=[
            pl.BlockSpec((BN, H), lambda i: (i, 0)),
            pl.BlockSpec((BN, H), lambda i: (i, 0)),
            pl.BlockSpec((BN, H), lambda i: (i, 0)),
        ],
        out_shape=[
            jax.ShapeDtypeStruct((N, H), _F32),
            jax.ShapeDtypeStruct((N, H), _F32),
            jax.ShapeDtypeStruct((N, H), _F32),
        ],
    )(x, wf, bf.reshape(1, H), wa, wb, bm.reshape(1, H))


def _node_mid_body(h, g0, g1, wa, wb, bm, h_out, a_out, b_out):
    h1 = h[...] + g0[...] + g1[...]
    h_out[...] = h1
    a_out[...] = jnp.dot(h1, wa[...], preferred_element_type=_F32)
    b_out[...] = jnp.dot(h1, wb[...], preferred_element_type=_F32) + bm[...]


def _node_mid(h, g0, g1, wa, wb, bm):
    return pl.pallas_call(
        _node_mid_body,
        grid=(N // BN,),
        in_specs=[
            pl.BlockSpec((BN, H), lambda i: (i, 0)),
            pl.BlockSpec((BN, H), lambda i: (i, 0)),
            pl.BlockSpec((BN, H), lambda i: (i, 0)),
            pl.BlockSpec((H, H), lambda i: (0, 0)),
            pl.BlockSpec((H, H), lambda i: (0, 0)),
            pl.BlockSpec((1, H), lambda i: (0, 0)),
        ],
        out_specs=[
            pl.BlockSpec((BN, H), lambda i: (i, 0)),
            pl.BlockSpec((BN, H), lambda i: (i, 0)),
            pl.BlockSpec((BN, H), lambda i: (i, 0)),
        ],
        out_shape=[
            jax.ShapeDtypeStruct((N, H), _F32),
            jax.ShapeDtypeStruct((N, H), _F32),
            jax.ShapeDtypeStruct((N, H), _F32),
        ],
    )(h, g0, g1, wa, wb, bm.reshape(1, H))


def _node_last_body(xres, h, g0, g1, wf, bf, out):
    h2 = h[...] + g0[...] + g1[...]
    out[...] = xres[...] + jnp.dot(h2, wf[...], preferred_element_type=_F32) + bf[...]


def _node_last(xres, h, g0, g1, wf, bf):
    return pl.pallas_call(
        _node_last_body,
        grid=(N // BN,),
        in_specs=[
            pl.BlockSpec((BN, D), lambda i: (i, 0)),
            pl.BlockSpec((BN, H), lambda i: (i, 0)),
            pl.BlockSpec((BN, H), lambda i: (i, 0)),
            pl.BlockSpec((BN, H), lambda i: (i, 0)),
            pl.BlockSpec((H, D), lambda i: (0, 0)),
            pl.BlockSpec((1, D), lambda i: (0, 0)),
        ],
        out_specs=pl.BlockSpec((BN, D), lambda i: (i, 0)),
        out_shape=jax.ShapeDtypeStruct((N, D), _F32),
    )(xres, h, g0, g1, wf, bf.reshape(1, D))


_sc_mesh = plsc.VectorSubcoreMesh(core_axis_name="c", subcore_axis_name="s")


@functools.partial(
    pl.kernel,
    out_type=jax.ShapeDtypeStruct((NCORE, N, H), _F32),
    mesh=_sc_mesh,
    scratch_types=[
        pltpu.VMEM((2, K), jnp.int32),
        pltpu.VMEM((2, K), jnp.int32),
        pltpu.VMEM((2, K, H), _F32),
        pltpu.VMEM((2, K, H), _F32),
        pltpu.VMEM((2, K, H // 2), jnp.int32),
        pltpu.VMEM_SHARED((N, H), _F32),
        pltpu.SemaphoreType.DMA,
        pltpu.SemaphoreType.DMA,
        pltpu.SemaphoreType.DMA,
        pltpu.SemaphoreType.DMA,
    ],
)
def _sc_mp(a_hbm, b_hbm, c_hbm, src_hbm, dst_hbm, out_hbm,
           srcv, dstv, buf_a, buf_b, buf_c, agg, sem0, sem1, semi0, semi1):
    cid = lax.axis_index("c")
    sid = lax.axis_index("s")
    wid = sid * NCORE + cid
    sems = (sem0, sem1)
    sems_i = (semi0, semi1)
    nfull = NBLK // NW
    extra = NBLK - nfull * NW

    zero = jnp.zeros((16,), _F32)

    def _zero_rows(r, carry):
        for v in range(H // 16):
            buf_a[0, r, pl.ds(v * 16, 16)] = zero
        return carry

    lax.fori_loop(0, K, _zero_rows, 0)
    base_r = sid * ROWS_PER_SUB
    nz = ROWS_PER_SUB // K
    for t in range(nz):
        pltpu.sync_copy(buf_a.at[0], agg.at[pl.ds(base_r + t * K, K)])
    if ROWS_PER_SUB > nz * K:
        pltpu.sync_copy(buf_a.at[0].at[pl.ds(0, ROWS_PER_SUB - nz * K)],
                        agg.at[pl.ds(base_r + nz * K, ROWS_PER_SUB - nz * K)])

    @pl.when(sid == NSUB - 1)
    def _zero_tail():
        pltpu.sync_copy(buf_a.at[0].at[pl.ds(0, N - NSUB * ROWS_PER_SUB)],
                        agg.at[pl.ds(NSUB * ROWS_PER_SUB, N - NSUB * ROWS_PER_SUB)])

    plsc.subcore_barrier()


    def _fire_idx(j, b):
        base = (wid + j * NW) * K
        pltpu.async_copy(src_hbm.at[pl.ds(base, K)], srcv.at[b], sems_i[b])
        pltpu.async_copy(dst_hbm.at[pl.ds(base, K)], dstv.at[b], sems_i[b])

    def _wait_idx(b):
        pltpu.make_async_copy(src_hbm.at[pl.ds(0, K)], srcv.at[b], sems_i[b]).wait()
        pltpu.make_async_copy(src_hbm.at[pl.ds(0, K)], dstv.at[b], sems_i[b]).wait()

    def _fire_gather(j, b):
        base = (wid + j * NW) * K
        pltpu.async_copy(a_hbm.at[srcv.at[b]], buf_a.at[b], sems[b])
        pltpu.async_copy(b_hbm.at[dstv.at[b]], buf_b.at[b], sems[b])
        pltpu.async_copy(c_hbm.at[pl.ds(base, K)], buf_c.at[b], sems[b])

    def _wait_in(b):
        pltpu.make_async_copy(a_hbm.at[pl.ds(0, K)], buf_a.at[b], sems[b]).wait()
        pltpu.make_async_copy(a_hbm.at[pl.ds(0, K)], buf_b.at[b], sems[b]).wait()
        pltpu.make_async_copy(c_hbm.at[pl.ds(0, K)], buf_c.at[b], sems[b]).wait()

    def _compute(b):
        hi_mask = jnp.int32(-65536)

        def _edge(e):
            for g in range(H // 32):
                cw = buf_c[b, e, pl.ds(16 * g, 16)]
                clo = lax.bitcast_convert_type(cw << 16, _F32)
                chi = lax.bitcast_convert_type(cw & hi_mask, _F32)
                for s, cc in ((0, clo), (1, chi)):
                    sl = pl.ds(32 * g + 16 * s, 16)
                    xv = buf_a[b, e, sl] + buf_b[b, e, sl] + cc
                    tneg = xv * (_mC1 + _mC2 * (xv * xv))
                    buf_a[b, e, sl] = xv / (1.0 + jnp.exp(tneg))

        plsc.parallel_loop(0, K, 1, unroll=2)(_edge)

    pltpu.sync_copy(src_hbm.at[pl.ds(wid * K, K)], srcv.at[0])
    pltpu.sync_copy(dst_hbm.at[pl.ds(wid * K, K)], dstv.at[0])
    _fire_gather(0, 0)

    @pl.when(wid + NW < NBLK)
    def _pro_idx():
        _fire_idx(1, 1)

    def _outer(i, carry):
        for b in range(2):
            j = 2 * i + b
            nxt = j + 1
            nb = (b + 1) % 2

            @pl.when(wid + nxt * NW < NBLK)
            def _prefetch():
                _wait_idx(nb)
                _fire_gather(nxt, nb)

            _wait_in(b)
            _compute(b)
            pltpu.sync_copy(buf_a.at[b], agg.at[dstv.at[b]], add=True)

            @pl.when(wid + (j + 2) * NW < NBLK)
            def _prefetch_idx():
                _fire_idx(j + 2, b)
        return carry

    lax.fori_loop(0, nfull // 2, _outer, 0)

    @pl.when(wid < extra)
    def _tail():
        _wait_in(0)
        _compute(0)
        pltpu.sync_copy(buf_a.at[0], agg.at[dstv.at[0]], add=True)

    plsc.subcore_barrier()
    pltpu.sync_copy(agg.at[pl.ds(base_r, ROWS_PER_SUB)],
                    out_hbm.at[cid, pl.ds(base_r, ROWS_PER_SUB)])

    @pl.when(sid == NSUB - 1)
    def _out_tail():
        pltpu.sync_copy(agg.at[pl.ds(NSUB * ROWS_PER_SUB, N - NSUB * ROWS_PER_SUB)],
                        out_hbm.at[cid, pl.ds(NSUB * ROWS_PER_SUB, N - NSUB * ROWS_PER_SUB)])


def kernel(x, edge_index, edge_attr, emb_W1, emb_b1, emb_W2, emb_b2,
           ff1_W, ff1_b, mp1_W, mp1_b, mp2_W, mp2_b, ff2_W, ff2_b):
    src = edge_index[0]
    dst = edge_index[1]

    ef = _compute_ef(edge_attr, emb_W1, emb_b1, emb_W2, emb_b2)

    c_terms = []
    for i in range(DEPTH):
        c_terms.append(_compute_c(ef, mp1_W[i, 2 * H:, :]))
        c_terms.append(_compute_c(ef, mp2_W[i, 2 * H:, :]))

    out = x
    for i in range(DEPTH):
        h, a, b = _node_first(out, ff1_W[i], ff1_b[i],
                              mp1_W[i, :H, :], mp1_W[i, H:2 * H, :], mp1_b[i])
        agg1 = _sc_mp(a, b, c_terms[2 * i], src, dst)
        h1, a2, b2 = _node_mid(h, agg1[0], agg1[1],
                               mp2_W[i, :H, :], mp2_W[i, H:2 * H, :], mp2_b[i])
        agg2 = _sc_mp(a2, b2, c_terms[2 * i + 1], src, dst)
        out = _node_last(out, h1, agg2[0], agg2[1], ff2_W[i], ff2_b[i])

    return (out, out)


# speedup vs baseline: 1.1191x; 1.1191x over previous
import functools

import jax
import jax.numpy as jnp
import numpy as np
from jax import lax
from jax.experimental import pallas as pl
from jax.experimental.pallas import tpu as pltpu
from jax.experimental.pallas import tpu_sc as plsc

N = 10000
E = 320000
D = 128
H = 128
EE = 16
DEPTH = 2

K = 64
NBLK = E // K
NCORE = 2
NSUB = 16
NW = NCORE * NSUB
ROWS_PER_SUB = 624

_F32 = jnp.float32
_BF16 = jnp.bfloat16
_C1 = 1.5957691216057308
_C2 = _C1 * 0.044715
_mC1 = -_C1
_mC2 = -_C2


BE = 2000


def _cfused_body(ea, w1, b1, w2, b2, wc, o0, o1, o2, o3):
    h = jax.nn.gelu(jnp.dot(ea[...], w1[...], preferred_element_type=_F32) + b1[...])
    ef = jnp.dot(h, w2[...], preferred_element_type=_F32) + b2[...]
    for j, out in enumerate((o0, o1, o2, o3)):
        out[...] = jnp.dot(ef, wc[j], preferred_element_type=_F32)


def _compute_c_terms(edge_attr, w1, b1, w2, b2, wc_all):
    cspec = pl.BlockSpec((BE, H), lambda i: (i, 0))
    cshape = jax.ShapeDtypeStruct((E, H), _F32)
    return pl.pallas_call(
        _cfused_body,
        grid=(E // BE,),
        in_specs=[
            pl.BlockSpec((BE, 4), lambda i: (i, 0)),
            pl.BlockSpec((4, EE), lambda i: (0, 0)),
            pl.BlockSpec((1, EE), lambda i: (0, 0)),
            pl.BlockSpec((EE, EE), lambda i: (0, 0)),
            pl.BlockSpec((1, EE), lambda i: (0, 0)),
            pl.BlockSpec((4, EE, H), lambda i: (0, 0, 0)),
        ],
        out_specs=[cspec, cspec, cspec, cspec],
        out_shape=[cshape, cshape, cshape, cshape],
    )(edge_attr, w1, b1.reshape(1, EE), w2, b2.reshape(1, EE), wc_all)


BN = 2000


def _node_first_body(x, wf, bf, wa, wb, bm, h_out, a_out, b_out):
    h = jax.nn.gelu(jnp.dot(x[...], wf[...], preferred_element_type=_F32) + bf[...])
    h_out[...] = h
    a_out[...] = jnp.dot(h, wa[...], preferred_element_type=_F32)
    b_out[...] = jnp.dot(h, wb[...], preferred_element_type=_F32) + bm[...]


def _node_first(x, wf, bf, wa, wb, bm):
    return pl.pallas_call(
        _node_first_body,
        grid=(N // BN,),
        in_specs=[
            pl.BlockSpec((BN, D), lambda i: (i, 0)),
            pl.BlockSpec((D, H), lambda i: (0, 0)),
            pl.BlockSpec((1, H), lambda i: (0, 0)),
            pl.BlockSpec((H, H), lambda i: (0, 0)),
            pl.BlockSpec((H, H), lambda i: (0, 0)),
            pl.BlockSpec((1, H), lambda i: (0, 0)),
        ],
        out_specs=[
            pl.BlockSpec((BN, H), lambda i: (i, 0)),
            pl.BlockSpec((BN, H), lambda i: (i, 0)),
            pl.BlockSpec((BN, H), lambda i: (i, 0)),
        ],
        out_shape=[
            jax.ShapeDtypeStruct((N, H), _F32),
            jax.ShapeDtypeStruct((N, H), _F32),
            jax.ShapeDtypeStruct((N, H), _F32),
        ],
    )(x, wf, bf.reshape(1, H), wa, wb, bm.reshape(1, H))


def _node_mid_body(h, g0, g1, wa, wb, bm, h_out, a_out, b_out):
    h1 = h[...] + g0[...] + g1[...]
    h_out[...] = h1
    a_out[...] = jnp.dot(h1, wa[...], preferred_element_type=_F32)
    b_out[...] = jnp.dot(h1, wb[...], preferred_element_type=_F32) + bm[...]


def _node_mid(h, g0, g1, wa, wb, bm):
    return pl.pallas_call(
        _node_mid_body,
        grid=(N // BN,),
        in_specs=[
            pl.BlockSpec((BN, H), lambda i: (i, 0)),
            pl.BlockSpec((BN, H), lambda i: (i, 0)),
            pl.BlockSpec((BN, H), lambda i: (i, 0)),
            pl.BlockSpec((H, H), lambda i: (0, 0)),
            pl.BlockSpec((H, H), lambda i: (0, 0)),
            pl.BlockSpec((1, H), lambda i: (0, 0)),
        ],
        out_specs=[
            pl.BlockSpec((BN, H), lambda i: (i, 0)),
            pl.BlockSpec((BN, H), lambda i: (i, 0)),
            pl.BlockSpec((BN, H), lambda i: (i, 0)),
        ],
        out_shape=[
            jax.ShapeDtypeStruct((N, H), _F32),
            jax.ShapeDtypeStruct((N, H), _F32),
            jax.ShapeDtypeStruct((N, H), _F32),
        ],
    )(h, g0, g1, wa, wb, bm.reshape(1, H))


def _node_last_body(xres, h, g0, g1, wf, bf, out):
    h2 = h[...] + g0[...] + g1[...]
    out[...] = xres[...] + jnp.dot(h2, wf[...], preferred_element_type=_F32) + bf[...]


def _node_last(xres, h, g0, g1, wf, bf):
    return pl.pallas_call(
        _node_last_body,
        grid=(N // BN,),
        in_specs=[
            pl.BlockSpec((BN, D), lambda i: (i, 0)),
            pl.BlockSpec((BN, H), lambda i: (i, 0)),
            pl.BlockSpec((BN, H), lambda i: (i, 0)),
            pl.BlockSpec((BN, H), lambda i: (i, 0)),
            pl.BlockSpec((H, D), lambda i: (0, 0)),
            pl.BlockSpec((1, D), lambda i: (0, 0)),
        ],
        out_specs=pl.BlockSpec((BN, D), lambda i: (i, 0)),
        out_shape=jax.ShapeDtypeStruct((N, D), _F32),
    )(xres, h, g0, g1, wf, bf.reshape(1, D))


_sc_mesh = plsc.VectorSubcoreMesh(core_axis_name="c", subcore_axis_name="s")


@functools.partial(
    pl.kernel,
    out_type=jax.ShapeDtypeStruct((NCORE, N, H), _F32),
    mesh=_sc_mesh,
    scratch_types=[
        pltpu.VMEM((2, K), jnp.int32),
        pltpu.VMEM((2, K), jnp.int32),
        pltpu.VMEM((2, K, H), _F32),
        pltpu.VMEM((2, K, H), _F32),
        pltpu.VMEM((2, K, H), _F32),
        pltpu.VMEM_SHARED((N, H), _F32),
        pltpu.SemaphoreType.DMA,
        pltpu.SemaphoreType.DMA,
        pltpu.SemaphoreType.DMA,
        pltpu.SemaphoreType.DMA,
    ],
)
def _sc_mp(a_hbm, b_hbm, c_hbm, src_hbm, dst_hbm, out_hbm,
           srcv, dstv, buf_a, buf_b, buf_c, agg, sem0, sem1, semi0, semi1):
    cid = lax.axis_index("c")
    sid = lax.axis_index("s")
    wid = sid * NCORE + cid
    sems = (sem0, sem1)
    sems_i = (semi0, semi1)
    nfull = NBLK // NW
    extra = NBLK - nfull * NW

    zero = jnp.zeros((16,), _F32)

    def _zero_rows(r, carry):
        for v in range(H // 16):
            buf_a[0, r, pl.ds(v * 16, 16)] = zero
        return carry

    lax.fori_loop(0, K, _zero_rows, 0)
    base_r = sid * ROWS_PER_SUB
    nz = ROWS_PER_SUB // K
    for t in range(nz):
        pltpu.sync_copy(buf_a.at[0], agg.at[pl.ds(base_r + t * K, K)])
    if ROWS_PER_SUB > nz * K:
        pltpu.sync_copy(buf_a.at[0].at[pl.ds(0, ROWS_PER_SUB - nz * K)],
                        agg.at[pl.ds(base_r + nz * K, ROWS_PER_SUB - nz * K)])

    @pl.when(sid == NSUB - 1)
    def _zero_tail():
        pltpu.sync_copy(buf_a.at[0].at[pl.ds(0, N - NSUB * ROWS_PER_SUB)],
                        agg.at[pl.ds(NSUB * ROWS_PER_SUB, N - NSUB * ROWS_PER_SUB)])

    plsc.subcore_barrier()


    def _fire_idx(j, b):
        base = (wid + j * NW) * K
        pltpu.async_copy(src_hbm.at[pl.ds(base, K)], srcv.at[b], sems_i[b])
        pltpu.async_copy(dst_hbm.at[pl.ds(base, K)], dstv.at[b], sems_i[b])

    def _wait_idx(b):
        pltpu.make_async_copy(src_hbm.at[pl.ds(0, K)], srcv.at[b], sems_i[b]).wait()
        pltpu.make_async_copy(src_hbm.at[pl.ds(0, K)], dstv.at[b], sems_i[b]).wait()

    def _fire_gather(j, b):
        base = (wid + j * NW) * K
        pltpu.async_copy(a_hbm.at[srcv.at[b]], buf_a.at[b], sems[b])
        pltpu.async_copy(b_hbm.at[dstv.at[b]], buf_b.at[b], sems[b])
        pltpu.async_copy(c_hbm.at[pl.ds(base, K)], buf_c.at[b], sems[b])

    def _wait_in(b):
        pltpu.make_async_copy(a_hbm.at[pl.ds(0, K)], buf_a.at[b], sems[b]).wait()
        pltpu.make_async_copy(a_hbm.at[pl.ds(0, K)], buf_b.at[b], sems[b]).wait()
        pltpu.make_async_copy(c_hbm.at[pl.ds(0, K)], buf_c.at[b], sems[b]).wait()

    def _compute(b):
        def _edge(e, c2):
            for v in range(H // 16):
                sl = pl.ds(16 * v, 16)
                xv = buf_a[b, e, sl] + buf_b[b, e, sl] + buf_c[b, e, sl]
                tneg = xv * (_mC1 + _mC2 * (xv * xv))
                buf_a[b, e, sl] = xv / (1.0 + jnp.exp(tneg))
            return c2

        lax.fori_loop(0, K, _edge, 0)

    pltpu.sync_copy(src_hbm.at[pl.ds(wid * K, K)], srcv.at[0])
    pltpu.sync_copy(dst_hbm.at[pl.ds(wid * K, K)], dstv.at[0])
    _fire_gather(0, 0)

    @pl.when(wid + NW < NBLK)
    def _pro_idx():
        _fire_idx(1, 1)

    def _outer(i, carry):
        for b in range(2):
            j = 2 * i + b
            nxt = j + 1
            nb = (b + 1) % 2

            @pl.when(wid + nxt * NW < NBLK)
            def _prefetch():
                _wait_idx(nb)
                _fire_gather(nxt, nb)

            _wait_in(b)
            _compute(b)
            pltpu.sync_copy(buf_a.at[b], agg.at[dstv.at[b]], add=True)

            @pl.when(wid + (j + 2) * NW < NBLK)
            def _prefetch_idx():
                _fire_idx(j + 2, b)
        return carry

    lax.fori_loop(0, nfull // 2, _outer, 0)

    @pl.when(wid < extra)
    def _tail():
        _wait_in(0)
        _compute(0)
        pltpu.sync_copy(buf_a.at[0], agg.at[dstv.at[0]], add=True)

    plsc.subcore_barrier()
    pltpu.sync_copy(agg.at[pl.ds(base_r, ROWS_PER_SUB)],
                    out_hbm.at[cid, pl.ds(base_r, ROWS_PER_SUB)])

    @pl.when(sid == NSUB - 1)
    def _out_tail():
        pltpu.sync_copy(agg.at[pl.ds(NSUB * ROWS_PER_SUB, N - NSUB * ROWS_PER_SUB)],
                        out_hbm.at[cid, pl.ds(NSUB * ROWS_PER_SUB, N - NSUB * ROWS_PER_SUB)])


def kernel(x, edge_index, edge_attr, emb_W1, emb_b1, emb_W2, emb_b2,
           ff1_W, ff1_b, mp1_W, mp1_b, mp2_W, mp2_b, ff2_W, ff2_b):
    src = edge_index[0]
    dst = edge_index[1]

    wc_all = jnp.stack([mp1_W[0, 2 * H:, :], mp2_W[0, 2 * H:, :],
                        mp1_W[1, 2 * H:, :], mp2_W[1, 2 * H:, :]])
    c_terms = list(_compute_c_terms(edge_attr, emb_W1, emb_b1,
                                    emb_W2, emb_b2, wc_all))

    out = x
    for i in range(DEPTH):
        h, a, b = _node_first(out, ff1_W[i], ff1_b[i],
                              mp1_W[i, :H, :], mp1_W[i, H:2 * H, :], mp1_b[i])
        agg1 = _sc_mp(a, b, c_terms[2 * i], src, dst)
        h1, a2, b2 = _node_mid(h, agg1[0], agg1[1],
                               mp2_W[i, :H, :], mp2_W[i, H:2 * H, :], mp2_b[i])
        agg2 = _sc_mp(a2, b2, c_terms[2 * i + 1], src, dst)
        out = _node_last(out, h1, agg2[0], agg2[1], ff2_W[i], ff2_b[i])

    return (out, out)


# speedup vs baseline: 1.1194x; 1.0002x over previous
import functools

import jax
import jax.numpy as jnp
from jax import lax
from jax.experimental import pallas as pl
from jax.experimental.pallas import tpu as pltpu
from jax.experimental.pallas import tpu_sc as plsc

N = 10000
E = 320000
D = 128
H = 128
EE = 16
DEPTH = 2

K = 64
NBLK = E // K
NCORE = 2
NSUB = 16
NW = NCORE * NSUB
ROWS_PER_SUB = 624

_F32 = jnp.float32
_C1 = 1.5957691216057308
_C2 = _C1 * 0.044715
_mC1 = -_C1
_mC2 = -_C2


BE = 2000


def _cfused_body(ea, w1, b1, w2, b2, wc, o0, o1, o2, o3):
    h = jax.nn.gelu(jnp.dot(ea[...], w1[...], preferred_element_type=_F32) + b1[...])
    ef = jnp.dot(h, w2[...], preferred_element_type=_F32) + b2[...]
    for j, out in enumerate((o0, o1, o2, o3)):
        out[...] = jnp.dot(ef, wc[j], preferred_element_type=_F32)


def _compute_c_terms(edge_attr, w1, b1, w2, b2, wc_all):
    cspec = pl.BlockSpec((BE, H), lambda i: (i, 0))
    cshape = jax.ShapeDtypeStruct((E, H), _F32)
    return pl.pallas_call(
        _cfused_body,
        grid=(E // BE,),
        in_specs=[
            pl.BlockSpec((BE, 4), lambda i: (i, 0)),
            pl.BlockSpec((4, EE), lambda i: (0, 0)),
            pl.BlockSpec((1, EE), lambda i: (0, 0)),
            pl.BlockSpec((EE, EE), lambda i: (0, 0)),
            pl.BlockSpec((1, EE), lambda i: (0, 0)),
            pl.BlockSpec((4, EE, H), lambda i: (0, 0, 0)),
        ],
        out_specs=[cspec, cspec, cspec, cspec],
        out_shape=[cshape, cshape, cshape, cshape],
    )(edge_attr, w1, b1.reshape(1, EE), w2, b2.reshape(1, EE), wc_all)


BN = 2000


def _node_first_body(x, wf, bf, wa, wb, bm, h_out, a_out, b_out):
    h = jax.nn.gelu(jnp.dot(x[...], wf[...], preferred_element_type=_F32) + bf[...])
    h_out[...] = h
    a_out[...] = jnp.dot(h, wa[...], preferred_element_type=_F32)
    b_out[...] = jnp.dot(h, wb[...], preferred_element_type=_F32) + bm[...]


def _node_first(x, wf, bf, wa, wb, bm):
    return pl.pallas_call(
        _node_first_body,
        grid=(N // BN,),
        in_specs=[
            pl.BlockSpec((BN, D), lambda i: (i, 0)),
            pl.BlockSpec((D, H), lambda i: (0, 0)),
            pl.BlockSpec((1, H), lambda i: (0, 0)),
            pl.BlockSpec((H, H), lambda i: (0, 0)),
            pl.BlockSpec((H, H), lambda i: (0, 0)),
            pl.BlockSpec((1, H), lambda i: (0, 0)),
        ],
        out_specs=[
            pl.BlockSpec((BN, H), lambda i: (i, 0)),
            pl.BlockSpec((BN, H), lambda i: (i, 0)),
            pl.BlockSpec((BN, H), lambda i: (i, 0)),
        ],
        out_shape=[
            jax.ShapeDtypeStruct((N, H), _F32),
            jax.ShapeDtypeStruct((N, H), _F32),
            jax.ShapeDtypeStruct((N, H), _F32),
        ],
    )(x, wf, bf.reshape(1, H), wa, wb, bm.reshape(1, H))


def _node_mid_body(h, g0, g1, wa, wb, bm, h_out, a_out, b_out):
    h1 = h[...] + g0[...] + g1[...]
    h_out[...] = h1
    a_out[...] = jnp.dot(h1, wa[...], preferred_element_type=_F32)
    b_out[...] = jnp.dot(h1, wb[...], preferred_element_type=_F32) + bm[...]


def _node_mid(h, g0, g1, wa, wb, bm):
    return pl.pallas_call(
        _node_mid_body,
        grid=(N // BN,),
        in_specs=[
            pl.BlockSpec((BN, H), lambda i: (i, 0)),
            pl.BlockSpec((BN, H), lambda i: (i, 0)),
            pl.BlockSpec((BN, H), lambda i: (i, 0)),
            pl.BlockSpec((H, H), lambda i: (0, 0)),
            pl.BlockSpec((H, H), lambda i: (0, 0)),
            pl.BlockSpec((1, H), lambda i: (0, 0)),
        ],
        out_specs=[
            pl.BlockSpec((BN, H), lambda i: (i, 0)),
            pl.BlockSpec((BN, H), lambda i: (i, 0)),
            pl.BlockSpec((BN, H), lambda i: (i, 0)),
        ],
        out_shape=[
            jax.ShapeDtypeStruct((N, H), _F32),
            jax.ShapeDtypeStruct((N, H), _F32),
            jax.ShapeDtypeStruct((N, H), _F32),
        ],
    )(h, g0, g1, wa, wb, bm.reshape(1, H))


def _node_last_body(xres, h, g0, g1, wf, bf, out):
    h2 = h[...] + g0[...] + g1[...]
    out[...] = xres[...] + jnp.dot(h2, wf[...], preferred_element_type=_F32) + bf[...]


def _node_last(xres, h, g0, g1, wf, bf):
    return pl.pallas_call(
        _node_last_body,
        grid=(N // BN,),
        in_specs=[
            pl.BlockSpec((BN, D), lambda i: (i, 0)),
            pl.BlockSpec((BN, H), lambda i: (i, 0)),
            pl.BlockSpec((BN, H), lambda i: (i, 0)),
            pl.BlockSpec((BN, H), lambda i: (i, 0)),
            pl.BlockSpec((H, D), lambda i: (0, 0)),
            pl.BlockSpec((1, D), lambda i: (0, 0)),
        ],
        out_specs=pl.BlockSpec((BN, D), lambda i: (i, 0)),
        out_shape=jax.ShapeDtypeStruct((N, D), _F32),
    )(xres, h, g0, g1, wf, bf.reshape(1, D))


_sc_mesh = plsc.VectorSubcoreMesh(core_axis_name="c", subcore_axis_name="s")


@functools.partial(
    pl.kernel,
    out_type=jax.ShapeDtypeStruct((NCORE, N, H), _F32),
    mesh=_sc_mesh,
    scratch_types=[
        pltpu.VMEM((2, K), jnp.int32),
        pltpu.VMEM((2, K), jnp.int32),
        pltpu.VMEM((2, K, H), _F32),
        pltpu.VMEM((2, K, H), _F32),
        pltpu.VMEM((2, K, H), _F32),
        pltpu.VMEM_SHARED((N, H), _F32),
        pltpu.SemaphoreType.DMA,
        pltpu.SemaphoreType.DMA,
        pltpu.SemaphoreType.DMA,
        pltpu.SemaphoreType.DMA,
    ],
)
def _sc_mp(a_hbm, b_hbm, c_hbm, src_hbm, dst_hbm, out_hbm,
           srcv, dstv, buf_a, buf_b, buf_c, agg, sem0, sem1, semi0, semi1):
    cid = lax.axis_index("c")
    sid = lax.axis_index("s")
    wid = sid * NCORE + cid
    sems = (sem0, sem1)
    sems_i = (semi0, semi1)
    nfull = NBLK // NW
    extra = NBLK - nfull * NW

    zero = jnp.zeros((16,), _F32)

    def _zero_rows(r, carry):
        for v in range(H // 16):
            buf_a[0, r, pl.ds(v * 16, 16)] = zero
        return carry

    lax.fori_loop(0, K, _zero_rows, 0)
    base_r = sid * ROWS_PER_SUB
    nz = ROWS_PER_SUB // K
    for t in range(nz):
        pltpu.sync_copy(buf_a.at[0], agg.at[pl.ds(base_r + t * K, K)])
    if ROWS_PER_SUB > nz * K:
        pltpu.sync_copy(buf_a.at[0].at[pl.ds(0, ROWS_PER_SUB - nz * K)],
                        agg.at[pl.ds(base_r + nz * K, ROWS_PER_SUB - nz * K)])

    @pl.when(sid == NSUB - 1)
    def _zero_tail():
        pltpu.sync_copy(buf_a.at[0].at[pl.ds(0, N - NSUB * ROWS_PER_SUB)],
                        agg.at[pl.ds(NSUB * ROWS_PER_SUB, N - NSUB * ROWS_PER_SUB)])

    plsc.subcore_barrier()


    def _fire_idx(j, b):
        base = (wid + j * NW) * K
        pltpu.async_copy(src_hbm.at[pl.ds(base, K)], srcv.at[b], sems_i[b])
        pltpu.async_copy(dst_hbm.at[pl.ds(base, K)], dstv.at[b], sems_i[b])

    def _wait_idx(b):
        pltpu.make_async_copy(src_hbm.at[pl.ds(0, K)], srcv.at[b], sems_i[b]).wait()
        pltpu.make_async_copy(src_hbm.at[pl.ds(0, K)], dstv.at[b], sems_i[b]).wait()

    def _fire_gather(j, b):
        base = (wid + j * NW) * K
        pltpu.async_copy(a_hbm.at[srcv.at[b]], buf_a.at[b], sems[b])
        pltpu.async_copy(b_hbm.at[dstv.at[b]], buf_b.at[b], sems[b])
        pltpu.async_copy(c_hbm.at[pl.ds(base, K)], buf_c.at[b], sems[b])

    def _wait_in(b):
        pltpu.make_async_copy(a_hbm.at[pl.ds(0, K)], buf_a.at[b], sems[b]).wait()
        pltpu.make_async_copy(a_hbm.at[pl.ds(0, K)], buf_b.at[b], sems[b]).wait()
        pltpu.make_async_copy(c_hbm.at[pl.ds(0, K)], buf_c.at[b], sems[b]).wait()

    def _compute(b):
        def _edge(e, c2):
            for v in range(H // 16):
                sl = pl.ds(16 * v, 16)
                xv = buf_a[b, e, sl] + buf_b[b, e, sl] + buf_c[b, e, sl]
                tneg = xv * (_mC1 + _mC2 * (xv * xv))
                buf_a[b, e, sl] = xv / (1.0 + jnp.exp(tneg))
            return c2

        lax.fori_loop(0, K, _edge, 0)

    pltpu.sync_copy(src_hbm.at[pl.ds(wid * K, K)], srcv.at[0])
    pltpu.sync_copy(dst_hbm.at[pl.ds(wid * K, K)], dstv.at[0])
    _fire_gather(0, 0)

    @pl.when(wid + NW < NBLK)
    def _pro_idx():
        _fire_idx(1, 1)

    def _outer(i, carry):
        for b in range(2):
            j = 2 * i + b
            nxt = j + 1
            nb = (b + 1) % 2

            @pl.when(wid + nxt * NW < NBLK)
            def _prefetch():
                _wait_idx(nb)
                _fire_gather(nxt, nb)

            _wait_in(b)
            _compute(b)
            pltpu.sync_copy(buf_a.at[b], agg.at[dstv.at[b]], add=True)

            @pl.when(wid + (j + 2) * NW < NBLK)
            def _prefetch_idx():
                _fire_idx(j + 2, b)
        return carry

    lax.fori_loop(0, nfull // 2, _outer, 0)

    @pl.when(wid < extra)
    def _tail():
        _wait_in(0)
        _compute(0)
        pltpu.sync_copy(buf_a.at[0], agg.at[dstv.at[0]], add=True)

    plsc.subcore_barrier()
    pltpu.sync_copy(agg.at[pl.ds(base_r, ROWS_PER_SUB)],
                    out_hbm.at[cid, pl.ds(base_r, ROWS_PER_SUB)])

    @pl.when(sid == NSUB - 1)
    def _out_tail():
        pltpu.sync_copy(agg.at[pl.ds(NSUB * ROWS_PER_SUB, N - NSUB * ROWS_PER_SUB)],
                        out_hbm.at[cid, pl.ds(NSUB * ROWS_PER_SUB, N - NSUB * ROWS_PER_SUB)])


def kernel(x, edge_index, edge_attr, emb_W1, emb_b1, emb_W2, emb_b2,
           ff1_W, ff1_b, mp1_W, mp1_b, mp2_W, mp2_b, ff2_W, ff2_b):
    src = edge_index[0]
    dst = edge_index[1]

    wc_all = jnp.stack([mp1_W[0, 2 * H:, :], mp2_W[0, 2 * H:, :],
                        mp1_W[1, 2 * H:, :], mp2_W[1, 2 * H:, :]])
    c_terms = list(_compute_c_terms(edge_attr, emb_W1, emb_b1,
                                    emb_W2, emb_b2, wc_all))

    out = x
    for i in range(DEPTH):
        h, a, b = _node_first(out, ff1_W[i], ff1_b[i],
                              mp1_W[i, :H, :], mp1_W[i, H:2 * H, :], mp1_b[i])
        agg1 = _sc_mp(a, b, c_terms[2 * i], src, dst)
        h1, a2, b2 = _node_mid(h, agg1[0], agg1[1],
                               mp2_W[i, :H, :], mp2_W[i, H:2 * H, :], mp2_b[i])
        agg2 = _sc_mp(a2, b2, c_terms[2 * i + 1], src, dst)
        out = _node_last(out, h1, agg2[0], agg2[1], ff2_W[i], ff2_b[i])

    return (out, out)
